# Initial kernel scaffold; baseline (speedup 1.0000x reference)
import jax
import jax.numpy as jnp
from jax.experimental import pallas as pl


def kernel(x, edge_index, edge_attr, memory, batch_id, Wn_s2t, We_s2t, Wm_s2t, bm_s2t, Wn_t2s, We_t2s, Wm_t2s, bm_t2s):
    raise NotImplementedError("write your pallas kernel here")


# speedup vs baseline: 3.7016x; 3.7016x over previous
import functools

import jax
import jax.numpy as jnp
from jax import lax
from jax.experimental import pallas as pl
from jax.experimental.pallas import tpu as pltpu
from jax.experimental.pallas import tpu_sc as plsc

NC = 2
NS = 16
K = 80
R = 2


def _sc_segment_sums(e0, e1, x, ea, z128, ones128):
    N, D = x.shape
    E = e0.shape[0]
    CPT = E // (NS * K)
    ITERS = CPT // R
    SLAB = 640
    STEP = 624
    assert (NS - 1) * STEP + SLAB == N and STEP % 8 == 0
    f32 = jnp.float32

    mesh = plsc.VectorSubcoreMesh(
        core_axis_name="c", subcore_axis_name="s", num_cores=NC, num_subcores=NS)

    @functools.partial(
        pl.kernel,
        out_type=[
            jax.ShapeDtypeStruct((N, D), f32),
            jax.ShapeDtypeStruct((N, D), f32),
            jax.ShapeDtypeStruct((N, D), f32),
            jax.ShapeDtypeStruct((N, D), f32),
            jax.ShapeDtypeStruct((N, D), f32),
            jax.ShapeDtypeStruct((N, D), f32),
        ],
        mesh=mesh,
        scratch_types=[
            pltpu.VMEM_SHARED((N, D), f32),
            [pltpu.VMEM((K,), jnp.int32) for _ in range(R)],
            [pltpu.VMEM((K,), jnp.int32) for _ in range(R)],
            pltpu.VMEM((R * K, D), f32),
            pltpu.SemaphoreType.DMA,
            pltpu.SemaphoreType.DMA,
        ],
    )
    def sc_kernel(e0_hbm, e1_hbm, x_hbm, ea_hbm, z128_hbm, ones_hbm,
                  xs1_out, xs2_out, ea1_out, ea2_out, dg1_out, dg2_out,
                  acc_sh, sidx, didx, rows, semg, sems):
        c = lax.axis_index("c")
        s = lax.axis_index("s")
        base = s * STEP
        chunkbase = s * CPT

        def zero_slab():
            pltpu.sync_copy(z128_hbm.at[pl.ds(base, SLAB)],
                            acc_sh.at[pl.ds(base, SLAB)])

        def copy_out(out):
            pltpu.sync_copy(acc_sh.at[pl.ds(base, SLAB)],
                            out.at[pl.ds(base, SLAB)])

        def run_direction(src_hbm, dst_hbm, xs_out, ea_out, dg_out):
            zero_slab()
            plsc.subcore_barrier()

            @pl.loop(0, ITERS)
            def _(j):
                for r in range(R):
                    off = (chunkbase + j * R + r) * K
                    pltpu.sync_copy(src_hbm.at[pl.ds(off, K)], sidx[r])
                    pltpu.sync_copy(dst_hbm.at[pl.ds(off, K)], didx[r])
                gets = [pltpu.async_copy(x_hbm.at[sidx[r]],
                                         rows.at[pl.ds(r * K, K)], semg)
                        for r in range(R)]
                for g in gets:
                    g.wait()
                puts = [pltpu.async_copy(rows.at[pl.ds(r * K, K)],
                                         acc_sh.at[didx[r]], sems, add=True)
                        for r in range(R)]
                for p in puts:
                    p.wait()

            plsc.subcore_barrier()
            copy_out(xs_out)
            plsc.subcore_barrier()
            zero_slab()
            plsc.subcore_barrier()

            @pl.loop(0, ITERS)
            def _(j):
                for r in range(R):
                    off = (chunkbase + j * R + r) * K
                    pltpu.sync_copy(dst_hbm.at[pl.ds(off, K)], didx[r])
                gets = [pltpu.async_copy(
                            ea_hbm.at[pl.ds((chunkbase + j * R + r) * K, K)],
                            rows.at[pl.ds(r * K, K)], semg)
                        for r in range(R)]
                for g in gets:
                    g.wait()
                puts = [pltpu.async_copy(rows.at[pl.ds(r * K, K)],
                                         acc_sh.at[didx[r]], sems, add=True)
                        for r in range(R)]
                for p in puts:
                    p.wait()

            plsc.subcore_barrier()
            copy_out(ea_out)
            plsc.subcore_barrier()
            zero_slab()
            pltpu.sync_copy(ones_hbm, rows.at[pl.ds(0, K)])
            plsc.subcore_barrier()

            @pl.loop(0, ITERS)
            def _(j):
                for r in range(R):
                    off = (chunkbase + j * R + r) * K
                    pltpu.sync_copy(dst_hbm.at[pl.ds(off, K)], didx[r])
                puts = [pltpu.async_copy(rows.at[pl.ds(0, K)],
                                         acc_sh.at[didx[r]], sems, add=True)
                        for r in range(R)]
                for p in puts:
                    p.wait()

            plsc.subcore_barrier()
            copy_out(dg_out)

        @pl.when(c == 0)
        def _():
            run_direction(e0_hbm, e1_hbm, xs1_out, ea1_out, dg1_out)

        @pl.when(c == 1)
        def _():
            run_direction(e1_hbm, e0_hbm, xs2_out, ea2_out, dg2_out)

    return sc_kernel(e0, e1, x, ea, z128, ones128)


def _combine_body(x_ref, xs1_ref, xs2_ref, ea1_ref, ea2_ref, d1_ref, d2_ref,
                  wn1_ref, we1_ref, wm1_ref, bm1_ref,
                  wn2_ref, we2_ref, wm2_ref, bm2_ref, o_ref):
    f32 = jnp.float32
    D = x_ref.shape[1]

    def mm_t(a, b):
        return lax.dot_general(a, b, (((1,), (1,)), ((), ())),
                               preferred_element_type=f32)

    xb = x_ref[...]
    wn1 = wn1_ref[...]
    wn2 = wn2_ref[...]
    wm1 = wm1_ref[...]
    wm2 = wm2_ref[...]
    d1 = d1_ref[:, 0:1]
    d2 = d2_ref[:, 0:1]
    agg = (mm_t(mm_t(xs1_ref[...], wn1), wm1[:, 0:D])
           + d1 * mm_t(mm_t(xb, wn1), wm1[:, D:2 * D])
           + mm_t(mm_t(ea1_ref[...], we1_ref[...]), wm1[:, 2 * D:3 * D])
           + d1 * bm1_ref[...]
           + mm_t(mm_t(xs2_ref[...], wn2), wm2[:, 0:D])
           + d2 * mm_t(mm_t(xb, wn2), wm2[:, D:2 * D])
           + mm_t(mm_t(ea2_ref[...], we2_ref[...]), wm2[:, 2 * D:3 * D])
           + d2 * bm2_ref[...])
    o_ref[...] = xb + 0.5 * agg


def _combine(x, xs1, xs2, ea1, ea2, dg1, dg2,
             Wn1, We1, Wm1, bm1, Wn2, We2, Wm2, bm2):
    N, D = x.shape
    NB = 1000
    grid = (N // NB,)
    row_spec = pl.BlockSpec((NB, D), lambda i: (i, 0))
    deg_spec = pl.BlockSpec((NB, D), lambda i: (i, 0))
    w_spec = pl.BlockSpec((D, D), lambda i: (0, 0))
    wm_spec = pl.BlockSpec((D, 3 * D), lambda i: (0, 0))
    b_spec = pl.BlockSpec((1, D), lambda i: (0, 0))
    return pl.pallas_call(
        _combine_body,
        grid=grid,
        in_specs=[row_spec, row_spec, row_spec, row_spec, row_spec,
                  deg_spec, deg_spec,
                  w_spec, w_spec, wm_spec, b_spec,
                  w_spec, w_spec, wm_spec, b_spec],
        out_specs=row_spec,
        out_shape=jax.ShapeDtypeStruct((N, D), jnp.float32),
    )(x, xs1, xs2, ea1, ea2, dg1, dg2,
      Wn1, We1, Wm1, bm1, Wn2, We2, Wm2, bm2)


def kernel(x, edge_index, edge_attr, memory, batch_id,
           Wn_s2t, We_s2t, Wm_s2t, bm_s2t,
           Wn_t2s, We_t2s, Wm_t2s, bm_t2s):
    N, D = x.shape
    E = edge_index.shape[1]
    assert E % (NS * K * R) == 0 and N % 1000 == 0

    z128 = jnp.zeros((N, D), jnp.float32)
    ones128 = jnp.ones((K, D), jnp.float32)

    xs1, xs2, ea1, ea2, dg1, dg2 = _sc_segment_sums(
        edge_index[0], edge_index[1], x, edge_attr, z128, ones128)

    out = _combine(x, xs1, xs2, ea1, ea2, dg1, dg2,
                   Wn_s2t, We_s2t, Wm_s2t, jnp.reshape(bm_s2t, (1, D)),
                   Wn_t2s, We_t2s, Wm_t2s, jnp.reshape(bm_t2s, (1, D)))
    return (out, edge_attr)


# speedup vs baseline: 5.8192x; 1.5721x over previous
import functools

import jax
import jax.numpy as jnp
from jax import lax
from jax.experimental import pallas as pl
from jax.experimental.pallas import tpu as pltpu
from jax.experimental.pallas import tpu_sc as plsc

NC = 2
NS = 16
K = 80
R = 2


def _sc_segment_sums(e0, e1, x, ea, z128, ones128):
    N, D = x.shape
    E = e0.shape[0]
    CPT = E // (NS * K)
    ITERS = CPT // R
    SLAB = 640
    STEP = 624
    assert (NS - 1) * STEP + SLAB == N and STEP % 8 == 0
    f32 = jnp.float32

    mesh = plsc.VectorSubcoreMesh(
        core_axis_name="c", subcore_axis_name="s", num_cores=NC, num_subcores=NS)

    @functools.partial(
        pl.kernel,
        out_type=[
            jax.ShapeDtypeStruct((N, D), f32),
            jax.ShapeDtypeStruct((N, D), f32),
            jax.ShapeDtypeStruct((N, D), f32),
            jax.ShapeDtypeStruct((N, D), f32),
            jax.ShapeDtypeStruct((N, D), f32),
            jax.ShapeDtypeStruct((N, D), f32),
        ],
        mesh=mesh,
        scratch_types=[
            pltpu.VMEM_SHARED((N, D), f32),
            [pltpu.VMEM((K,), jnp.int32) for _ in range(4)],
            [pltpu.VMEM((K,), jnp.int32) for _ in range(4)],
            pltpu.VMEM((4 * K, D), f32),
            pltpu.SemaphoreType.DMA,
            pltpu.SemaphoreType.DMA,
            pltpu.SemaphoreType.DMA,
        ],
    )
    def sc_kernel(e0_hbm, e1_hbm, x_hbm, ea_hbm, z128_hbm, ones_hbm,
                  xs1_out, xs2_out, ea1_out, ea2_out, dg1_out, dg2_out,
                  acc_sh, sidx, didx, rows, semi, semg, sems):
        c = lax.axis_index("c")
        s = lax.axis_index("s")
        base = s * STEP
        chunkbase = s * CPT
        NPAIRS = CPT // 2
        assert NPAIRS % 2 == 1

        def zero_slab():
            pltpu.sync_copy(z128_hbm.at[pl.ds(base, SLAB)],
                            acc_sh.at[pl.ds(base, SLAB)])

        def copy_out(out):
            pltpu.sync_copy(acc_sh.at[pl.ds(base, SLAB)],
                            out.at[pl.ds(base, SLAB)])

        def run_phase(mode, src_hbm, dst_hbm):
            def off(P, r):
                return (chunkbase + 2 * P + r) * K

            def idx_copies(P, sp):
                cps = []
                for r in range(2):
                    if mode == "gather":
                        cps.append((src_hbm.at[pl.ds(off(P, r), K)],
                                    sidx[2 * sp + r]))
                    cps.append((dst_hbm.at[pl.ds(off(P, r), K)],
                                didx[2 * sp + r]))
                return cps

            def rows_copies(P, sp):
                cps = []
                for r in range(2):
                    sl = rows.at[pl.ds((2 * sp + r) * K, K)]
                    if mode == "gather":
                        cps.append((x_hbm.at[sidx[2 * sp + r]], sl))
                    elif mode == "seq":
                        cps.append((ea_hbm.at[pl.ds(off(P, r), K)], sl))
                return cps

            def scat_copies(sp):
                cps = []
                for r in range(2):
                    src = (rows.at[pl.ds(0, K)] if mode == "const"
                           else rows.at[pl.ds((2 * sp + r) * K, K)])
                    cps.append((src, acc_sh.at[didx[2 * sp + r]]))
                return cps

            def fire(cps, sem, add=False):
                for a, b in cps:
                    pltpu.async_copy(a, b, sem, add=add)

            def drain(cps, sem):
                for a, b in cps:
                    pltpu.make_async_copy(a, b, sem).wait()

            def body(P, sp, first):
                if mode != "const":
                    drain(rows_copies(P, sp), semg)
                fire(scat_copies(sp), sems, add=True)
                if not first:
                    drain(scat_copies(1 - sp), sems)
                fire(idx_copies(P + 1, 1 - sp), semi)
                drain(idx_copies(P + 1, 1 - sp), semi)
                if mode != "const":
                    fire(rows_copies(P + 1, 1 - sp), semg)

            if mode == "const":
                pltpu.sync_copy(ones_hbm, rows.at[pl.ds(0, K)])
            fire(idx_copies(0, 0), semi)
            drain(idx_copies(0, 0), semi)
            if mode != "const":
                fire(rows_copies(0, 0), semg)
            body(0, 0, first=True)

            @pl.loop(0, (NPAIRS - 3) // 2)
            def _(j2):
                P = 2 * j2 + 1
                body(P, 1, first=False)
                body(P + 1, 0, first=False)

            body(NPAIRS - 2, 1, first=False)
            P = NPAIRS - 1
            if mode != "const":
                drain(rows_copies(P, 0), semg)
            fire(scat_copies(0), sems, add=True)
            drain(scat_copies(1), sems)
            drain(scat_copies(0), sems)

        def run_direction(src_hbm, dst_hbm, xs_out, ea_out, dg_out):
            zero_slab()
            plsc.subcore_barrier()
            run_phase("gather", src_hbm, dst_hbm)
            plsc.subcore_barrier()
            copy_out(xs_out)
            plsc.subcore_barrier()
            zero_slab()
            plsc.subcore_barrier()
            run_phase("seq", src_hbm, dst_hbm)
            plsc.subcore_barrier()
            copy_out(ea_out)
            plsc.subcore_barrier()
            zero_slab()
            plsc.subcore_barrier()
            run_phase("const", src_hbm, dst_hbm)
            plsc.subcore_barrier()
            copy_out(dg_out)

        @pl.when(c == 0)
        def _():
            run_direction(e0_hbm, e1_hbm, xs1_out, ea1_out, dg1_out)

        @pl.when(c == 1)
        def _():
            run_direction(e1_hbm, e0_hbm, xs2_out, ea2_out, dg2_out)

    return sc_kernel(e0, e1, x, ea, z128, ones128)


def _combine_body(x_ref, xs1_ref, xs2_ref, ea1_ref, ea2_ref, d1_ref, d2_ref,
                  wn1_ref, we1_ref, wm1_ref, bm1_ref,
                  wn2_ref, we2_ref, wm2_ref, bm2_ref, o_ref):
    f32 = jnp.float32
    D = x_ref.shape[1]

    def mm_t(a, b):
        return lax.dot_general(a, b, (((1,), (1,)), ((), ())),
                               preferred_element_type=f32)

    xb = x_ref[...]
    wn1 = wn1_ref[...]
    wn2 = wn2_ref[...]
    wm1 = wm1_ref[...]
    wm2 = wm2_ref[...]
    d1 = d1_ref[:, 0:1]
    d2 = d2_ref[:, 0:1]
    agg = (mm_t(mm_t(xs1_ref[...], wn1), wm1[:, 0:D])
           + d1 * mm_t(mm_t(xb, wn1), wm1[:, D:2 * D])
           + mm_t(mm_t(ea1_ref[...], we1_ref[...]), wm1[:, 2 * D:3 * D])
           + d1 * bm1_ref[...]
           + mm_t(mm_t(xs2_ref[...], wn2), wm2[:, 0:D])
           + d2 * mm_t(mm_t(xb, wn2), wm2[:, D:2 * D])
           + mm_t(mm_t(ea2_ref[...], we2_ref[...]), wm2[:, 2 * D:3 * D])
           + d2 * bm2_ref[...])
    o_ref[...] = xb + 0.5 * agg


def _combine(x, xs1, xs2, ea1, ea2, dg1, dg2,
             Wn1, We1, Wm1, bm1, Wn2, We2, Wm2, bm2):
    N, D = x.shape
    NB = 1000
    grid = (N // NB,)
    row_spec = pl.BlockSpec((NB, D), lambda i: (i, 0))
    deg_spec = pl.BlockSpec((NB, D), lambda i: (i, 0))
    w_spec = pl.BlockSpec((D, D), lambda i: (0, 0))
    wm_spec = pl.BlockSpec((D, 3 * D), lambda i: (0, 0))
    b_spec = pl.BlockSpec((1, D), lambda i: (0, 0))
    return pl.pallas_call(
        _combine_body,
        grid=grid,
        in_specs=[row_spec, row_spec, row_spec, row_spec, row_spec,
                  deg_spec, deg_spec,
                  w_spec, w_spec, wm_spec, b_spec,
                  w_spec, w_spec, wm_spec, b_spec],
        out_specs=row_spec,
        out_shape=jax.ShapeDtypeStruct((N, D), jnp.float32),
    )(x, xs1, xs2, ea1, ea2, dg1, dg2,
      Wn1, We1, Wm1, bm1, Wn2, We2, Wm2, bm2)


def kernel(x, edge_index, edge_attr, memory, batch_id,
           Wn_s2t, We_s2t, Wm_s2t, bm_s2t,
           Wn_t2s, We_t2s, Wm_t2s, bm_t2s):
    N, D = x.shape
    E = edge_index.shape[1]
    assert E % (NS * K * R) == 0 and N % 1000 == 0

    z128 = jnp.zeros((N, D), jnp.float32)
    ones128 = jnp.ones((K, D), jnp.float32)

    xs1, xs2, ea1, ea2, dg1, dg2 = _sc_segment_sums(
        edge_index[0], edge_index[1], x, edge_attr, z128, ones128)

    out = _combine(x, xs1, xs2, ea1, ea2, dg1, dg2,
                   Wn_s2t, We_s2t, Wm_s2t, jnp.reshape(bm_s2t, (1, D)),
                   Wn_t2s, We_t2s, Wm_t2s, jnp.reshape(bm_t2s, (1, D)))
    return (out, edge_attr)
